# Initial kernel scaffold; baseline (speedup 1.0000x reference)
import jax
import jax.numpy as jnp
from jax.experimental import pallas as pl


def kernel(final_boxes, final_scores, final_classes, descriptors, places_db):
    raise NotImplementedError("write your pallas kernel here")


# speedup vs baseline: 1.0720x; 1.0720x over previous
import functools

import jax
import jax.numpy as jnp
from jax import lax
from jax.experimental import pallas as pl
from jax.experimental.pallas import tpu as pltpu
from jax.experimental.pallas import tpu_sc as plsc

_TOPK = 5
_MIN_SIM = 0.8
_MIN_VOTES = 0.0
_NQ = 64
_DIM = 64
_NDB = 100000
_BLK = 5000
_NBLOCKS = _NDB // _BLK

_NEG_INF = float("-inf")
_PAD_IDX = 1e9


def _tc_body(desc_ref, db_ref, sims_ref, idx_ref, rs_ref, ri_ref):
    i = pl.program_id(0)

    @pl.when(i == 0)
    def _init():
        rs_ref[...] = jnp.full((_NQ, 128), _NEG_INF, jnp.float32)
        ri_ref[...] = jnp.full((_NQ, 128), _PAD_IDX, jnp.float32)

    desc = desc_ref[...]
    x = db_ref[:, :_DIM]
    sims = jax.lax.dot_general(
        desc, x, (((1,), (1,)), ((), ())),
        preferred_element_type=jnp.float32)

    lane = jax.lax.broadcasted_iota(jnp.int32, sims.shape, 1)
    iota128 = jax.lax.broadcasted_iota(jnp.int32, (_NQ, 128), 1)

    c_s = jnp.full((_NQ, 128), _NEG_INF, jnp.float32)
    c_i = jnp.full((_NQ, 128), _PAD_IDX, jnp.float32)
    for j in range(_TOPK):
        m = jnp.max(sims, axis=1, keepdims=True)
        eq = sims == m
        pos = jnp.min(jnp.where(eq, lane, 2 ** 30), axis=1, keepdims=True)
        posf = (pos + i * _BLK).astype(jnp.float32)
        c_s = jnp.where(iota128 == 8 + j, m, c_s)
        c_i = jnp.where(iota128 == 8 + j, posf, c_i)
        if j < _TOPK - 1:
            sims = jnp.where(lane == pos, _NEG_INF, sims)

    comb_s = jnp.where(iota128 < 8, rs_ref[...], c_s)
    comb_i = jnp.where(iota128 < 8, ri_ref[...], c_i)
    n_s = jnp.full((_NQ, 128), _NEG_INF, jnp.float32)
    n_i = jnp.full((_NQ, 128), _PAD_IDX, jnp.float32)
    for j in range(_TOPK):
        m = jnp.max(comb_s, axis=1, keepdims=True)
        eq = comb_s == m
        pidx = jnp.min(jnp.where(eq, comb_i, _PAD_IDX), axis=1, keepdims=True)
        oh = eq & (comb_i == pidx)
        n_s = jnp.where(iota128 == j, m, n_s)
        n_i = jnp.where(iota128 == j, pidx, n_i)
        comb_s = jnp.where(oh, _NEG_INF, comb_s)
    rs_ref[...] = n_s
    ri_ref[...] = n_i

    @pl.when(i == _NBLOCKS - 1)
    def _finish():
        sims_ref[...] = n_s
        idx_ref[...] = jnp.where(iota128 < _TOPK, n_i, 0.0).astype(jnp.int32)


def _tc_topk(descriptors, places_db):
    return pl.pallas_call(
        _tc_body,
        grid=(_NBLOCKS,),
        in_specs=[
            pl.BlockSpec((_NQ, _DIM), lambda i: (0, 0)),
            pl.BlockSpec((_BLK, _DIM + 1), lambda i: (i, 0)),
        ],
        out_specs=[
            pl.BlockSpec((_NQ, 128), lambda i: (0, 0)),
            pl.BlockSpec((_NQ, 128), lambda i: (0, 0)),
        ],
        out_shape=[
            jax.ShapeDtypeStruct((_NQ, 128), jnp.float32),
            jax.ShapeDtypeStruct((_NQ, 128), jnp.int32),
        ],
        scratch_shapes=[
            pltpu.VMEM((_NQ, 128), jnp.float32),
            pltpu.VMEM((_NQ, 128), jnp.float32),
        ],
        compiler_params=pltpu.CompilerParams(
            dimension_semantics=("arbitrary",)),
    )(descriptors, places_db)


def _sc_vote(top_sims, top_idx, flat_db):
    mesh = plsc.VectorSubcoreMesh(core_axis_name="c", subcore_axis_name="s")

    @functools.partial(
        pl.kernel,
        mesh=mesh,
        out_type=[
            jax.ShapeDtypeStruct((_NQ, 16), jnp.float32),
            jax.ShapeDtypeStruct((_NQ, 16), jnp.int32),
        ],
        scratch_types=[
            pltpu.VMEM((16,), jnp.float32),
            pltpu.VMEM((16,), jnp.int32),
            pltpu.VMEM((16,), jnp.float32),
            pltpu.VMEM((16,), jnp.float32),
            pltpu.VMEM((16,), jnp.int32),
            pltpu.SemaphoreType.DMA,
        ],
    )
    def k(sims_hbm, idx_hbm, db_hbm, so_hbm, co_hbm,
          s_v, i_v, p_v, so_v, co_v, sem):
        cid = lax.axis_index("c")
        sid = lax.axis_index("s")
        wid = sid * 2 + cid
        for t in range(2):
            q = wid * 2 + t
            pltpu.sync_copy(sims_hbm.at[q, pl.ds(0, 16)], s_v)
            pltpu.sync_copy(idx_hbm.at[q, pl.ds(0, 16)], i_v)
            flat = i_v[...] * (_DIM + 1) + _DIM
            pltpu.async_copy(db_hbm.at[flat], p_v, sem).wait()

            lane16 = lax.iota(jnp.int32, 16)
            p = p_v[...]
            sv = s_v[...]
            lane_ok = lane16 < _TOPK

            def _gat(x, idx):
                return x.at[idx].get(mode="promise_in_bounds")

            def _bcast(x, j):
                return _gat(x, jnp.full((16,), j, jnp.int32))

            def _amax(x):
                for kk in (1, 2, 4, 8):
                    x = jnp.maximum(x, _gat(x, lane16 ^ kk))
                return x

            def _asum(x):
                for kk in (1, 2, 4, 8):
                    x = x + _gat(x, lane16 ^ kk)
                return x

            valid = (sv >= _MIN_SIM) & lane_ok
            validf = jnp.where(valid, 1.0, 0.0).astype(jnp.float32)
            counts = jnp.zeros((16,), jnp.float32)
            for j in range(_TOPK):
                pj = _bcast(p, j)
                vj = _bcast(validf, j)
                counts = counts + jnp.where(p == pj, vj, 0.0)
            score = jnp.where(valid, counts * 1e6 - p, _NEG_INF)
            mscore = _amax(score)
            ohm = score == mscore
            maj = _amax(jnp.where(ohm, p, -2e9))
            majc = _amax(jnp.where(ohm, counts, -1.0))
            nval = _asum(validf)
            anyv = nval > 0.0
            ratio = majc / jnp.maximum(nval, 1.0)
            acc = anyv & (ratio >= _MIN_VOTES)
            clsv = jnp.where(acc, maj, -1.0)
            match = (p == maj) & lane_ok
            smatch = _amax(jnp.where(match, sv, _NEG_INF))
            soutv = jnp.where(acc, smatch, 0.0)
            lane0 = lane16 == 0
            so_v[...] = jnp.where(lane0, soutv, 0.0)
            co_v[...] = jnp.where(lane0, clsv, 0.0).astype(jnp.int32)
            pltpu.sync_copy(so_v, so_hbm.at[q])
            pltpu.sync_copy(co_v, co_hbm.at[q])

    return k(top_sims, top_idx, flat_db)


def kernel(final_boxes, final_scores, final_classes, descriptors, places_db):
    top_sims, top_idx = _tc_topk(descriptors, places_db)
    flat_db = places_db.reshape(-1)
    scores16, classes16 = _sc_vote(top_sims, top_idx, flat_db)
    return final_boxes, scores16[:, 0], classes16[:, 0]


# speedup vs baseline: 1.4207x; 1.3252x over previous
import functools

import jax
import jax.numpy as jnp
from jax import lax
from jax.experimental import pallas as pl
from jax.experimental.pallas import tpu as pltpu
from jax.experimental.pallas import tpu_sc as plsc

_TOPK = 5
_MIN_SIM = 0.8
_MIN_VOTES = 0.0
_NQ = 64
_DIM = 64
_NDB = 100000
_BLK = 5000
_NBLOCKS = _NDB // _BLK

_NEG_INF = float("-inf")
_PAD_IDX = 1e9


def _tc_body(desc_ref, db_ref, sims_ref, idx_ref, rs_ref, ri_ref):
    i = pl.program_id(0)

    @pl.when(i == 0)
    def _init():
        rs_ref[...] = jnp.full((_NQ, 128), _NEG_INF, jnp.float32)
        ri_ref[...] = jnp.full((_NQ, 128), _PAD_IDX, jnp.float32)

    desc = desc_ref[...]
    x = db_ref[:, :_DIM]
    sims = jax.lax.dot_general(
        desc, x, (((1,), (1,)), ((), ())),
        preferred_element_type=jnp.float32)

    lane = jax.lax.broadcasted_iota(jnp.int32, sims.shape, 1)
    iota128 = jax.lax.broadcasted_iota(jnp.int32, (_NQ, 128), 1)

    c_s = jnp.full((_NQ, 128), _NEG_INF, jnp.float32)
    c_i = jnp.full((_NQ, 128), _PAD_IDX, jnp.float32)
    for j in range(_TOPK):
        m = jnp.max(sims, axis=1, keepdims=True)
        eq = sims == m
        pos = jnp.min(jnp.where(eq, lane, 2 ** 30), axis=1, keepdims=True)
        posf = (pos + i * _BLK).astype(jnp.float32)
        c_s = jnp.where(iota128 == 8 + j, m, c_s)
        c_i = jnp.where(iota128 == 8 + j, posf, c_i)
        if j < _TOPK - 1:
            sims = jnp.where(lane == pos, _NEG_INF, sims)

    comb_s = jnp.where(iota128 < 8, rs_ref[...], c_s)
    comb_i = jnp.where(iota128 < 8, ri_ref[...], c_i)
    n_s = jnp.full((_NQ, 128), _NEG_INF, jnp.float32)
    n_i = jnp.full((_NQ, 128), _PAD_IDX, jnp.float32)
    for j in range(_TOPK):
        m = jnp.max(comb_s, axis=1, keepdims=True)
        eq = comb_s == m
        pidx = jnp.min(jnp.where(eq, comb_i, _PAD_IDX), axis=1, keepdims=True)
        oh = eq & (comb_i == pidx)
        n_s = jnp.where(iota128 == j, m, n_s)
        n_i = jnp.where(iota128 == j, pidx, n_i)
        comb_s = jnp.where(oh, _NEG_INF, comb_s)
    rs_ref[...] = n_s
    ri_ref[...] = n_i

    @pl.when(i == _NBLOCKS - 1)
    def _finish():
        sims_ref[...] = n_s
        idx_ref[...] = jnp.where(iota128 < _TOPK, n_i, 0.0).astype(jnp.int32)


def _tc_topk(descriptors, places_db):
    return pl.pallas_call(
        _tc_body,
        grid=(_NBLOCKS,),
        in_specs=[
            pl.BlockSpec((_NQ, _DIM), lambda i: (0, 0)),
            pl.BlockSpec((_BLK, _DIM + 1), lambda i: (i, 0)),
        ],
        out_specs=[
            pl.BlockSpec((_NQ, 128), lambda i: (0, 0)),
            pl.BlockSpec((_NQ, 128), lambda i: (0, 0)),
        ],
        out_shape=[
            jax.ShapeDtypeStruct((_NQ, 128), jnp.float32),
            jax.ShapeDtypeStruct((_NQ, 128), jnp.int32),
        ],
        scratch_shapes=[
            pltpu.VMEM((_NQ, 128), jnp.float32),
            pltpu.VMEM((_NQ, 128), jnp.float32),
        ],
        compiler_params=pltpu.CompilerParams(
            dimension_semantics=("arbitrary",)),
    )(descriptors, places_db)


def _sc_vote(top_sims, top_idx, ids_arr):
    mesh = plsc.VectorSubcoreMesh(core_axis_name="c", subcore_axis_name="s")

    @functools.partial(
        pl.kernel,
        mesh=mesh,
        out_type=[
            jax.ShapeDtypeStruct((_NQ, 16), jnp.float32),
            jax.ShapeDtypeStruct((_NQ, 16), jnp.int32),
        ],
        scratch_types=[
            pltpu.VMEM((16,), jnp.float32),
            pltpu.VMEM((16,), jnp.int32),
            pltpu.VMEM((16,), jnp.float32),
            pltpu.VMEM((16,), jnp.float32),
            pltpu.VMEM((16,), jnp.int32),
            pltpu.SemaphoreType.DMA,
        ],
    )
    def k(sims_hbm, idx_hbm, db_hbm, so_hbm, co_hbm,
          s_v, i_v, p_v, so_v, co_v, sem):
        cid = lax.axis_index("c")
        sid = lax.axis_index("s")
        wid = sid * 2 + cid
        for t in range(2):
            q = wid * 2 + t
            pltpu.sync_copy(sims_hbm.at[q, pl.ds(0, 16)], s_v)
            pltpu.sync_copy(idx_hbm.at[q, pl.ds(0, 16)], i_v)
            pltpu.async_copy(db_hbm.at[i_v], p_v, sem).wait()

            lane16 = lax.iota(jnp.int32, 16)
            p = p_v[...]
            sv = s_v[...]
            lane_ok = lane16 < _TOPK

            def _gat(x, idx):
                return x.at[idx].get(mode="promise_in_bounds")

            def _bcast(x, j):
                return _gat(x, jnp.full((16,), j, jnp.int32))

            def _amax(x):
                for kk in (1, 2, 4, 8):
                    x = jnp.maximum(x, _gat(x, lane16 ^ kk))
                return x

            def _asum(x):
                for kk in (1, 2, 4, 8):
                    x = x + _gat(x, lane16 ^ kk)
                return x

            valid = (sv >= _MIN_SIM) & lane_ok
            validf = jnp.where(valid, 1.0, 0.0).astype(jnp.float32)
            counts = jnp.zeros((16,), jnp.float32)
            for j in range(_TOPK):
                pj = _bcast(p, j)
                vj = _bcast(validf, j)
                counts = counts + jnp.where(p == pj, vj, 0.0)
            score = jnp.where(valid, counts * 1e6 - p, _NEG_INF)
            mscore = _amax(score)
            ohm = score == mscore
            maj = _amax(jnp.where(ohm, p, -2e9))
            majc = _amax(jnp.where(ohm, counts, -1.0))
            nval = _asum(validf)
            anyv = nval > 0.0
            ratio = majc / jnp.maximum(nval, 1.0)
            acc = anyv & (ratio >= _MIN_VOTES)
            clsv = jnp.where(acc, maj, -1.0)
            match = (p == maj) & lane_ok
            smatch = _amax(jnp.where(match, sv, _NEG_INF))
            soutv = jnp.where(acc, smatch, 0.0)
            lane0 = lane16 == 0
            so_v[...] = jnp.where(lane0, soutv, 0.0)
            co_v[...] = jnp.where(lane0, clsv, 0.0).astype(jnp.int32)
            pltpu.sync_copy(so_v, so_hbm.at[q])
            pltpu.sync_copy(co_v, co_hbm.at[q])

    return k(top_sims, top_idx, ids_arr)


def kernel(final_boxes, final_scores, final_classes, descriptors, places_db):
    top_sims, top_idx = _tc_topk(descriptors, places_db)
    ids_arr = places_db[:, _DIM]
    scores16, classes16 = _sc_vote(top_sims, top_idx, ids_arr)
    return final_boxes, scores16[:, 0], classes16[:, 0]


# speedup vs baseline: 1.6814x; 1.1835x over previous
import functools

import jax
import jax.numpy as jnp
from jax import lax
from jax.experimental import pallas as pl
from jax.experimental.pallas import tpu as pltpu
from jax.experimental.pallas import tpu_sc as plsc

_TOPK = 5
_MIN_SIM = 0.8
_MIN_VOTES = 0.0
_NQ = 64
_DIM = 64
_NDB = 100000
_BLK = 10000
_NBLOCKS = _NDB // _BLK

_NEG_INF = float("-inf")
_PAD_IDX = 1e9


def _tc_body(desc_ref, db_ref, sims_ref, idx_ref, rs_ref, ri_ref):
    i = pl.program_id(0)

    @pl.when(i == 0)
    def _init():
        rs_ref[...] = jnp.full((_NQ, 128), _NEG_INF, jnp.float32)
        ri_ref[...] = jnp.full((_NQ, 128), _PAD_IDX, jnp.float32)

    desc = desc_ref[...]
    x = db_ref[:, :_DIM]
    sims = jax.lax.dot_general(
        desc, x, (((1,), (1,)), ((), ())),
        preferred_element_type=jnp.float32)

    lane = jax.lax.broadcasted_iota(jnp.int32, sims.shape, 1)
    iota128 = jax.lax.broadcasted_iota(jnp.int32, (_NQ, 128), 1)

    c_s = rs_ref[...]
    c_i = ri_ref[...]
    for j in range(_TOPK):
        m = jnp.max(sims, axis=1, keepdims=True)
        eq = sims == m
        pos = jnp.min(jnp.where(eq, lane, 2 ** 30), axis=1, keepdims=True)
        posf = (pos + i * _BLK).astype(jnp.float32)
        c_s = jnp.where(iota128 == i * _TOPK + j, m, c_s)
        c_i = jnp.where(iota128 == i * _TOPK + j, posf, c_i)
        if j < _TOPK - 1:
            sims = jnp.where(lane == pos, _NEG_INF, sims)
    rs_ref[...] = c_s
    ri_ref[...] = c_i

    @pl.when(i == _NBLOCKS - 1)
    def _finish():
        comb_s = c_s
        comb_i = c_i
        n_s = jnp.full((_NQ, 128), _NEG_INF, jnp.float32)
        n_i = jnp.full((_NQ, 128), _PAD_IDX, jnp.float32)
        for j in range(_TOPK):
            m = jnp.max(comb_s, axis=1, keepdims=True)
            eq = comb_s == m
            pidx = jnp.min(jnp.where(eq, comb_i, _PAD_IDX),
                           axis=1, keepdims=True)
            oh = eq & (comb_i == pidx)
            n_s = jnp.where(iota128 == j, m, n_s)
            n_i = jnp.where(iota128 == j, pidx, n_i)
            comb_s = jnp.where(oh, _NEG_INF, comb_s)
        sims_ref[...] = n_s
        idx_ref[...] = jnp.where(iota128 < _TOPK, n_i, 0.0).astype(jnp.int32)


def _tc_topk(descriptors, places_db):
    return pl.pallas_call(
        _tc_body,
        grid=(_NBLOCKS,),
        in_specs=[
            pl.BlockSpec((_NQ, _DIM), lambda i: (0, 0)),
            pl.BlockSpec((_BLK, _DIM + 1), lambda i: (i, 0)),
        ],
        out_specs=[
            pl.BlockSpec((_NQ, 128), lambda i: (0, 0)),
            pl.BlockSpec((_NQ, 128), lambda i: (0, 0)),
        ],
        out_shape=[
            jax.ShapeDtypeStruct((_NQ, 128), jnp.float32),
            jax.ShapeDtypeStruct((_NQ, 128), jnp.int32),
        ],
        scratch_shapes=[
            pltpu.VMEM((_NQ, 128), jnp.float32),
            pltpu.VMEM((_NQ, 128), jnp.float32),
        ],
        compiler_params=pltpu.CompilerParams(
            dimension_semantics=("arbitrary",)),
    )(descriptors, places_db)


def _sc_vote(top_sims, top_idx, ids_arr):
    mesh = plsc.VectorSubcoreMesh(core_axis_name="c", subcore_axis_name="s")

    @functools.partial(
        pl.kernel,
        mesh=mesh,
        out_type=[
            jax.ShapeDtypeStruct((_NQ, 16), jnp.float32),
            jax.ShapeDtypeStruct((_NQ, 16), jnp.int32),
        ],
        scratch_types=[
            pltpu.VMEM((16,), jnp.float32),
            pltpu.VMEM((16,), jnp.int32),
            pltpu.VMEM((16,), jnp.float32),
            pltpu.VMEM((16,), jnp.float32),
            pltpu.VMEM((16,), jnp.int32),
            pltpu.SemaphoreType.DMA,
        ],
    )
    def k(sims_hbm, idx_hbm, db_hbm, so_hbm, co_hbm,
          s_v, i_v, p_v, so_v, co_v, sem):
        cid = lax.axis_index("c")
        sid = lax.axis_index("s")
        wid = sid * 2 + cid
        for t in range(2):
            q = wid * 2 + t
            pltpu.sync_copy(sims_hbm.at[q, pl.ds(0, 16)], s_v)
            pltpu.sync_copy(idx_hbm.at[q, pl.ds(0, 16)], i_v)
            pltpu.async_copy(db_hbm.at[i_v], p_v, sem).wait()

            lane16 = lax.iota(jnp.int32, 16)
            p = p_v[...]
            sv = s_v[...]
            lane_ok = lane16 < _TOPK

            def _gat(x, idx):
                return x.at[idx].get(mode="promise_in_bounds")

            def _bcast(x, j):
                return _gat(x, jnp.full((16,), j, jnp.int32))

            def _amax(x):
                for kk in (1, 2, 4, 8):
                    x = jnp.maximum(x, _gat(x, lane16 ^ kk))
                return x

            def _asum(x):
                for kk in (1, 2, 4, 8):
                    x = x + _gat(x, lane16 ^ kk)
                return x

            valid = (sv >= _MIN_SIM) & lane_ok
            validf = jnp.where(valid, 1.0, 0.0).astype(jnp.float32)
            counts = jnp.zeros((16,), jnp.float32)
            for j in range(_TOPK):
                pj = _bcast(p, j)
                vj = _bcast(validf, j)
                counts = counts + jnp.where(p == pj, vj, 0.0)
            score = jnp.where(valid, counts * 1e6 - p, _NEG_INF)
            mscore = _amax(score)
            ohm = score == mscore
            maj = _amax(jnp.where(ohm, p, -2e9))
            majc = _amax(jnp.where(ohm, counts, -1.0))
            nval = _asum(validf)
            anyv = nval > 0.0
            ratio = majc / jnp.maximum(nval, 1.0)
            acc = anyv & (ratio >= _MIN_VOTES)
            clsv = jnp.where(acc, maj, -1.0)
            match = (p == maj) & lane_ok
            smatch = _amax(jnp.where(match, sv, _NEG_INF))
            soutv = jnp.where(acc, smatch, 0.0)
            lane0 = lane16 == 0
            so_v[...] = jnp.where(lane0, soutv, 0.0)
            co_v[...] = jnp.where(lane0, clsv, 0.0).astype(jnp.int32)
            pltpu.sync_copy(so_v, so_hbm.at[q])
            pltpu.sync_copy(co_v, co_hbm.at[q])

    return k(top_sims, top_idx, ids_arr)


def kernel(final_boxes, final_scores, final_classes, descriptors, places_db):
    top_sims, top_idx = _tc_topk(descriptors, places_db)
    ids_arr = places_db[:, _DIM]
    scores16, classes16 = _sc_vote(top_sims, top_idx, ids_arr)
    return final_boxes, scores16[:, 0], classes16[:, 0]


# speedup vs baseline: 1.7088x; 1.0163x over previous
import functools

import jax
import jax.numpy as jnp
from jax import lax
from jax.experimental import pallas as pl
from jax.experimental.pallas import tpu as pltpu
from jax.experimental.pallas import tpu_sc as plsc

_TOPK = 5
_MIN_SIM = 0.8
_MIN_VOTES = 0.0
_NQ = 64
_DIM = 64
_NDB = 100000
_BLK = 20000
_NBLOCKS = _NDB // _BLK

_NEG_INF = float("-inf")
_PAD_IDX = 1e9


def _tc_body(desc_ref, db_ref, sims_ref, idx_ref, rs_ref, ri_ref):
    i = pl.program_id(0)

    @pl.when(i == 0)
    def _init():
        rs_ref[...] = jnp.full((_NQ, 128), _NEG_INF, jnp.float32)
        ri_ref[...] = jnp.full((_NQ, 128), _PAD_IDX, jnp.float32)

    desc = desc_ref[...]
    x = db_ref[:, :_DIM]
    sims = jax.lax.dot_general(
        desc, x, (((1,), (1,)), ((), ())),
        preferred_element_type=jnp.float32)

    lane = jax.lax.broadcasted_iota(jnp.int32, sims.shape, 1)
    iota128 = jax.lax.broadcasted_iota(jnp.int32, (_NQ, 128), 1)

    c_s = rs_ref[...]
    c_i = ri_ref[...]
    for j in range(_TOPK):
        m = jnp.max(sims, axis=1, keepdims=True)
        eq = sims == m
        pos = jnp.min(jnp.where(eq, lane, 2 ** 30), axis=1, keepdims=True)
        posf = (pos + i * _BLK).astype(jnp.float32)
        c_s = jnp.where(iota128 == i * _TOPK + j, m, c_s)
        c_i = jnp.where(iota128 == i * _TOPK + j, posf, c_i)
        if j < _TOPK - 1:
            sims = jnp.where(lane == pos, _NEG_INF, sims)
    rs_ref[...] = c_s
    ri_ref[...] = c_i

    @pl.when(i == _NBLOCKS - 1)
    def _finish():
        comb_s = c_s
        comb_i = c_i
        n_s = jnp.full((_NQ, 128), _NEG_INF, jnp.float32)
        n_i = jnp.full((_NQ, 128), _PAD_IDX, jnp.float32)
        for j in range(_TOPK):
            m = jnp.max(comb_s, axis=1, keepdims=True)
            eq = comb_s == m
            pidx = jnp.min(jnp.where(eq, comb_i, _PAD_IDX),
                           axis=1, keepdims=True)
            oh = eq & (comb_i == pidx)
            n_s = jnp.where(iota128 == j, m, n_s)
            n_i = jnp.where(iota128 == j, pidx, n_i)
            comb_s = jnp.where(oh, _NEG_INF, comb_s)
        sims_ref[...] = n_s
        idx_ref[...] = jnp.where(iota128 < _TOPK, n_i, 0.0).astype(jnp.int32)


def _tc_topk(descriptors, places_db):
    return pl.pallas_call(
        _tc_body,
        grid=(_NBLOCKS,),
        in_specs=[
            pl.BlockSpec((_NQ, _DIM), lambda i: (0, 0)),
            pl.BlockSpec((_BLK, _DIM + 1), lambda i: (i, 0)),
        ],
        out_specs=[
            pl.BlockSpec((_NQ, 128), lambda i: (0, 0)),
            pl.BlockSpec((_NQ, 128), lambda i: (0, 0)),
        ],
        out_shape=[
            jax.ShapeDtypeStruct((_NQ, 128), jnp.float32),
            jax.ShapeDtypeStruct((_NQ, 128), jnp.int32),
        ],
        scratch_shapes=[
            pltpu.VMEM((_NQ, 128), jnp.float32),
            pltpu.VMEM((_NQ, 128), jnp.float32),
        ],
        compiler_params=pltpu.CompilerParams(
            dimension_semantics=("arbitrary",)),
    )(descriptors, places_db)


def _sc_vote(top_sims, top_idx, ids_arr):
    mesh = plsc.VectorSubcoreMesh(core_axis_name="c", subcore_axis_name="s")

    @functools.partial(
        pl.kernel,
        mesh=mesh,
        out_type=[
            jax.ShapeDtypeStruct((_NQ, 16), jnp.float32),
            jax.ShapeDtypeStruct((_NQ, 16), jnp.int32),
        ],
        scratch_types=[
            pltpu.VMEM((16,), jnp.float32),
            pltpu.VMEM((16,), jnp.int32),
            pltpu.VMEM((16,), jnp.float32),
            pltpu.VMEM((16,), jnp.float32),
            pltpu.VMEM((16,), jnp.int32),
            pltpu.SemaphoreType.DMA,
        ],
    )
    def k(sims_hbm, idx_hbm, db_hbm, so_hbm, co_hbm,
          s_v, i_v, p_v, so_v, co_v, sem):
        cid = lax.axis_index("c")
        sid = lax.axis_index("s")
        wid = sid * 2 + cid
        for t in range(2):
            q = wid * 2 + t
            pltpu.sync_copy(sims_hbm.at[q, pl.ds(0, 16)], s_v)
            pltpu.sync_copy(idx_hbm.at[q, pl.ds(0, 16)], i_v)
            pltpu.async_copy(db_hbm.at[i_v], p_v, sem).wait()

            lane16 = lax.iota(jnp.int32, 16)
            p = p_v[...]
            sv = s_v[...]
            lane_ok = lane16 < _TOPK

            def _gat(x, idx):
                return x.at[idx].get(mode="promise_in_bounds")

            def _bcast(x, j):
                return _gat(x, jnp.full((16,), j, jnp.int32))

            def _amax(x):
                for kk in (1, 2, 4, 8):
                    x = jnp.maximum(x, _gat(x, lane16 ^ kk))
                return x

            def _asum(x):
                for kk in (1, 2, 4, 8):
                    x = x + _gat(x, lane16 ^ kk)
                return x

            valid = (sv >= _MIN_SIM) & lane_ok
            validf = jnp.where(valid, 1.0, 0.0).astype(jnp.float32)
            counts = jnp.zeros((16,), jnp.float32)
            for j in range(_TOPK):
                pj = _bcast(p, j)
                vj = _bcast(validf, j)
                counts = counts + jnp.where(p == pj, vj, 0.0)
            score = jnp.where(valid, counts * 1e6 - p, _NEG_INF)
            mscore = _amax(score)
            ohm = score == mscore
            maj = _amax(jnp.where(ohm, p, -2e9))
            majc = _amax(jnp.where(ohm, counts, -1.0))
            nval = _asum(validf)
            anyv = nval > 0.0
            ratio = majc / jnp.maximum(nval, 1.0)
            acc = anyv & (ratio >= _MIN_VOTES)
            clsv = jnp.where(acc, maj, -1.0)
            match = (p == maj) & lane_ok
            smatch = _amax(jnp.where(match, sv, _NEG_INF))
            soutv = jnp.where(acc, smatch, 0.0)
            lane0 = lane16 == 0
            so_v[...] = jnp.where(lane0, soutv, 0.0)
            co_v[...] = jnp.where(lane0, clsv, 0.0).astype(jnp.int32)
            pltpu.sync_copy(so_v, so_hbm.at[q])
            pltpu.sync_copy(co_v, co_hbm.at[q])

    return k(top_sims, top_idx, ids_arr)


def kernel(final_boxes, final_scores, final_classes, descriptors, places_db):
    top_sims, top_idx = _tc_topk(descriptors, places_db)
    ids_arr = places_db[:, _DIM]
    scores16, classes16 = _sc_vote(top_sims, top_idx, ids_arr)
    return final_boxes, scores16[:, 0], classes16[:, 0]
